# baseline (speedup 1.0000x reference)
import jax
import jax.numpy as jnp
from jax.experimental import pallas as pl
from jax.experimental.pallas import tpu as pltpu

_FEAT = 30
_LB = 131072


def _affine_tanh_kernel(x_ref, w_ref, c_ref, o_ref):
    y = jnp.sum(x_ref[...] * w_ref[...], axis=0, keepdims=True)
    o_ref[...] = jnp.tanh(y + c_ref[0])


def kernel(features, w1, b1, w2, b2, w3, b3):
    B = features.shape[0]
    x_t = features.astype(jnp.float32).T

    w_col = (w3 @ w2 @ w1).reshape(_FEAT, 1).astype(jnp.float32)
    c = (b1 @ w2.T @ w3.T + b2 @ w3.T + b3).reshape(1).astype(jnp.float32)

    if B > _LB:
        lb = _LB
    elif B >= 256:
        lb = ((B // 2 + 127) // 128) * 128
    else:
        lb = B
    num_tiles = pl.cdiv(B, lb)

    out = pl.pallas_call(
        _affine_tanh_kernel,
        out_shape=jax.ShapeDtypeStruct((1, B), jnp.float32),
        grid=(num_tiles,),
        in_specs=[
            pl.BlockSpec((_FEAT, lb), lambda i: (0, i)),
            pl.BlockSpec((_FEAT, 1), lambda i: (0, 0)),
            pl.BlockSpec(memory_space=pltpu.MemorySpace.SMEM),
        ],
        out_specs=pl.BlockSpec((1, lb), lambda i: (0, i)),
        compiler_params=pltpu.CompilerParams(
            dimension_semantics=("parallel",),
        ),
    )(x_t, w_col, c)

    return out.reshape(B, 1)


# speedup vs baseline: 1.0488x; 1.0488x over previous
import jax
import jax.numpy as jnp
from jax.experimental import pallas as pl
from jax.experimental.pallas import tpu as pltpu

_FEAT = 30
_LB = 65536


def _affine_tanh_kernel(x_ref, w_ref, c_ref, o_ref):
    y = jnp.sum(x_ref[...] * w_ref[...], axis=0, keepdims=True)
    o_ref[...] = jnp.tanh(y + c_ref[0])


def kernel(features, w1, b1, w2, b2, w3, b3):
    B = features.shape[0]
    x_t = features.astype(jnp.float32).T

    w_col = (w3 @ w2 @ w1).reshape(_FEAT, 1).astype(jnp.float32)
    c = (b1 @ w2.T @ w3.T + b2 @ w3.T + b3).reshape(1).astype(jnp.float32)

    if B > _LB:
        lb = _LB
    elif B >= 256:
        lb = ((B // 2 + 127) // 128) * 128
    else:
        lb = B
    num_tiles = pl.cdiv(B, lb)

    out = pl.pallas_call(
        _affine_tanh_kernel,
        out_shape=jax.ShapeDtypeStruct((1, B), jnp.float32),
        grid=(num_tiles,),
        in_specs=[
            pl.BlockSpec((_FEAT, lb), lambda i: (0, i)),
            pl.BlockSpec((_FEAT, 1), lambda i: (0, 0)),
            pl.BlockSpec(memory_space=pltpu.MemorySpace.SMEM),
        ],
        out_specs=pl.BlockSpec((1, lb), lambda i: (0, i)),
        compiler_params=pltpu.CompilerParams(
            dimension_semantics=("parallel",),
        ),
    )(x_t, w_col, c)

    return out.reshape(B, 1)


# speedup vs baseline: 1.5777x; 1.5043x over previous
import jax
import jax.numpy as jnp
from jax import lax
from jax.experimental import pallas as pl
from jax.experimental.pallas import tpu as pltpu

_FEAT = 30
_LB = 65536
_F32 = jnp.float32
_H1, _H2 = 16, 8


def _fold_tanh_kernel(x_ref, w1_ref, w2_ref, w3_ref,
                      w2s_ref, w3s_ref, b1s_ref, b2s_ref, b3s_ref, o_ref):
    std = (((1,), (0,)), ((), ()))
    t = lax.dot_general(w3_ref[...], w2_ref[...], std,
                        preferred_element_type=_F32)
    wr = lax.dot_general(t, w1_ref[...], std,
                         preferred_element_type=_F32)
    y = lax.dot_general(wr, x_ref[...], std,
                        preferred_element_type=_F32)

    c = b3s_ref[0]
    for m in range(_H2):
        h = b2s_ref[m]
        for j in range(_H1):
            h = h + w2s_ref[m, j] * b1s_ref[j]
        c = c + w3s_ref[0, m] * h

    o_ref[...] = jnp.tanh(y + c)


def kernel(features, w1, b1, w2, b2, w3, b3):
    B = features.shape[0]
    x_t = features.astype(_F32).T

    if B > _LB:
        lb = _LB
    elif B >= 256:
        lb = ((B // 2 + 127) // 128) * 128
    else:
        lb = B
    num_tiles = pl.cdiv(B, lb)

    full = lambda shape: pl.BlockSpec(shape, lambda i: tuple(0 for _ in shape))
    smem = pl.BlockSpec(memory_space=pltpu.MemorySpace.SMEM)
    w1f, w2f, w3f = w1.astype(_F32), w2.astype(_F32), w3.astype(_F32)
    out = pl.pallas_call(
        _fold_tanh_kernel,
        out_shape=jax.ShapeDtypeStruct((1, B), _F32),
        grid=(num_tiles,),
        in_specs=[
            pl.BlockSpec((_FEAT, lb), lambda i: (0, i)),
            full((_H1, _FEAT)),
            full((_H2, _H1)),
            full((1, _H2)),
            smem, smem, smem, smem, smem,
        ],
        out_specs=pl.BlockSpec((1, lb), lambda i: (0, i)),
        compiler_params=pltpu.CompilerParams(
            dimension_semantics=("parallel",),
        ),
    )(x_t, w1f, w2f, w3f,
      w2f, w3f, b1.astype(_F32), b2.astype(_F32), b3.astype(_F32))

    return out.reshape(B, 1)
